# baseline (speedup 1.0000x reference)
import functools

import jax
import jax.numpy as jnp
from jax import lax
from jax.experimental import pallas as pl
from jax.experimental.pallas import tpu as pltpu
from jax.experimental.pallas import tpu_sc as plsc

N_NODES = 10000
NPAD = 10240
D = 256
DH = 128
E = 160000
K = 100
ROWS = E // K
NS = 16
TROWS = ROWS // NS
EPT = E // NS
NPT = N_NODES // NS
NBUF = 5
CROWS = NPAD // DH
OFFS = ((0, 100), (100, 100), (200, 100), (300, 100), (400, 100),
        (500, 100), (600, 25))


def _sc_segment_sum(x0, x1, src2, dst2, dst1):
    mesh = plsc.VectorSubcoreMesh(core_axis_name="c", subcore_axis_name="s")

    @functools.partial(
        pl.kernel,
        mesh=mesh,
        compiler_params=pltpu.CompilerParams(use_tc_tiling_on_sc=False,
                                             needs_layout_passes=False),
        out_type=(
            jax.ShapeDtypeStruct((N_NODES, DH), jnp.bfloat16),
            jax.ShapeDtypeStruct((N_NODES, DH), jnp.bfloat16),
            jax.ShapeDtypeStruct((CROWS, DH), jnp.float32),
        ),
        scratch_types=[
            pltpu.VMEM((TROWS, K), jnp.int32),
            pltpu.VMEM((TROWS, K), jnp.int32),
            pltpu.VMEM((EPT,), jnp.int32),
            [pltpu.VMEM((K, DH), jnp.bfloat16) for _ in range(NBUF)],
            pltpu.VMEM((CROWS, DH), jnp.float32),
            pltpu.VMEM((CROWS,), jnp.int32),
            pltpu.VMEM_SHARED((N_NODES, DH), jnp.bfloat16),
            pltpu.VMEM_SHARED((CROWS, DH), jnp.float32),
            [pltpu.SemaphoreType.DMA for _ in range(NBUF)],
            [pltpu.SemaphoreType.DMA for _ in range(NBUF)],
            pltpu.SemaphoreType.DMA,
            pltpu.SemaphoreType.DMA,
        ],
    )
    def k(x0_hbm, x1_hbm, src_hbm, dst_hbm, dst1_hbm, out0, out1, cnt_out,
          sidx, didx, dch, bufs, blk, riota, acc, cnt_sp, semg, sems,
          semi, semw):
        c = lax.axis_index("c")
        s = lax.axis_index("s")

        zero16 = jnp.zeros((16,), jnp.float32)
        one16 = jnp.ones((16,), jnp.float32)
        zero32 = jnp.zeros((32,), jnp.bfloat16)

        pltpu.async_copy(src_hbm.at[pl.ds(s * TROWS, TROWS)], sidx, semi)
        pltpu.async_copy(dst_hbm.at[pl.ds(s * TROWS, TROWS)], didx, semi)
        pltpu.async_copy(dst1_hbm.at[pl.ds(s * EPT, EPT)], dch, semi)

        rows0 = bufs[0]

        def zrow(i, carry):
            for j in range(DH // 32):
                rows0[i, pl.ds(j * 32, 32)] = zero32
            return carry

        lax.fori_loop(0, K, zrow, 0)

        def brow(i, carry):
            for j in range(DH // 16):
                blk[i, pl.ds(j * 16, 16)] = zero16
            return carry

        lax.fori_loop(0, CROWS, brow, 0)

        for j in range(CROWS // 16):
            riota[pl.ds(j * 16, 16)] = lax.iota(jnp.int32, 16) + j * 16

        for o, l in OFFS:
            pltpu.async_copy(rows0.at[pl.ds(0, l)],
                             acc.at[pl.ds(s * NPT + o, l)], semw)

        @pl.when(s == 0)
        def _():
            pltpu.async_copy(blk, cnt_sp, semw)

        for o, l in OFFS:
            pltpu.make_async_copy(rows0.at[pl.ds(0, l)],
                                  acc.at[pl.ds(s * NPT + o, l)], semw).wait()

        @pl.when(s == 0)
        def _():
            pltpu.make_async_copy(blk, cnt_sp, semw).wait()

        pltpu.make_async_copy(src_hbm.at[pl.ds(s * TROWS, TROWS)], sidx,
                              semi).wait()
        pltpu.make_async_copy(dst_hbm.at[pl.ds(s * TROWS, TROWS)], didx,
                              semi).wait()
        pltpu.make_async_copy(dst1_hbm.at[pl.ds(s * EPT, EPT)], dch,
                              semi).wait()

        plsc.subcore_barrier()

        def edge_loop(x_ref):
            for i in range(NBUF):
                pltpu.async_copy(x_ref.at[sidx.at[i]], bufs[i], semg[i])

            def cbody(j, carry):
                idx = dch[pl.ds(j * 16, 16)]
                plsc.addupdate_scatter(
                    blk,
                    [lax.shift_right_logical(idx, 7),
                     lax.bitwise_and(idx, 127)],
                    one16,
                )
                return carry

            lax.fori_loop(0, EPT // 16, cbody, 0)

            def step(q, carry):
                for i in range(NBUF):
                    a = NBUF * q + i
                    j = (i - 1) % NBUF
                    b = a - 1
                    pltpu.make_async_copy(x_ref.at[sidx.at[a]], bufs[i],
                                          semg[i]).wait()

                    @pl.when(b >= 0)
                    def _():
                        pltpu.make_async_copy(bufs[j], acc.at[didx.at[b]],
                                              sems[j]).wait()

                    pltpu.async_copy(bufs[i], acc.at[didx.at[a]], sems[i],
                                     add=True)

                    @pl.when(jnp.logical_and(b >= 0, b + NBUF < TROWS))
                    def _():
                        pltpu.async_copy(x_ref.at[sidx.at[b + NBUF]], bufs[j],
                                         semg[j])
                return carry

            lax.fori_loop(0, TROWS // NBUF, step, 0)

            i = (TROWS - 1) % NBUF
            pltpu.make_async_copy(bufs[i], acc.at[didx.at[TROWS - 1]],
                                  sems[i]).wait()

        @pl.when(c == 0)
        def _():
            edge_loop(x0_hbm)

        @pl.when(c == 1)
        def _():
            edge_loop(x1_hbm)

        plsc.subcore_barrier()

        pltpu.sync_copy(blk, cnt_sp.at[riota], add=True)
        plsc.subcore_barrier()

        def copy_out(dref):
            for n, (o, l) in enumerate(OFFS):
                buf = bufs[n % 2]
                if n >= 2:
                    po, pll = OFFS[n - 2]
                    pltpu.make_async_copy(
                        buf.at[pl.ds(0, pll)],
                        dref.at[pl.ds(s * NPT + po, pll)], semw).wait()
                pltpu.sync_copy(acc.at[pl.ds(s * NPT + o, l)],
                                buf.at[pl.ds(0, l)])
                pltpu.async_copy(buf.at[pl.ds(0, l)],
                                 dref.at[pl.ds(s * NPT + o, l)], semw)
            for n in (len(OFFS) - 2, len(OFFS) - 1):
                o, l = OFFS[n]
                buf = bufs[n % 2]
                pltpu.make_async_copy(buf.at[pl.ds(0, l)],
                                      dref.at[pl.ds(s * NPT + o, l)],
                                      semw).wait()

        @pl.when(c == 0)
        def _():
            copy_out(out0)

        @pl.when(c == 1)
        def _():
            copy_out(out1)

        @pl.when(jnp.logical_and(c == 0, s == 0))
        def _():
            pltpu.sync_copy(cnt_sp, blk)
            pltpu.sync_copy(blk, cnt_out)

    return k(x0, x1, src2, dst2, dst1)


BM = 2000


def _tc_body(x_ref, nb0_ref, nb1_ref, cnt_ref, wst_ref, wnt0_ref, wnt1_ref,
             b_ref, g_ref, be_ref, o_ref):
    hs = jnp.dot(x_ref[...], wst_ref[...], preferred_element_type=jnp.float32)
    hn = (jnp.dot(nb0_ref[...].astype(jnp.float32), wnt0_ref[...],
                  preferred_element_type=jnp.float32)
          + jnp.dot(nb1_ref[...].astype(jnp.float32), wnt1_ref[...],
                    preferred_element_type=jnp.float32))
    inv = 1.0 / jnp.maximum(cnt_ref[...], 1.0)
    h = hs + hn * inv + b_ref[...]
    mu = jnp.mean(h, axis=-1, keepdims=True)
    d = h - mu
    var = jnp.mean(d * d, axis=-1, keepdims=True)
    o_ref[...] = d * lax.rsqrt(var + 1e-5) * g_ref[...] + be_ref[...]


def _tc_combine(x, nb0, nb1, cnt, wst, wnt0, wnt1, bias, gamma, beta):
    grid = (N_NODES // BM,)
    return pl.pallas_call(
        _tc_body,
        grid=grid,
        compiler_params=pltpu.CompilerParams(
            allow_input_fusion=[True] * 10),
        in_specs=[
            pl.BlockSpec((BM, D), lambda i: (i, 0)),
            pl.BlockSpec((BM, DH), lambda i: (i, 0)),
            pl.BlockSpec((BM, DH), lambda i: (i, 0)),
            pl.BlockSpec((BM, 1), lambda i: (i, 0)),
            pl.BlockSpec((D, D), lambda i: (0, 0)),
            pl.BlockSpec((DH, D), lambda i: (0, 0)),
            pl.BlockSpec((DH, D), lambda i: (0, 0)),
            pl.BlockSpec((1, D), lambda i: (0, 0)),
            pl.BlockSpec((1, D), lambda i: (0, 0)),
            pl.BlockSpec((1, D), lambda i: (0, 0)),
        ],
        out_specs=pl.BlockSpec((BM, D), lambda i: (i, 0)),
        out_shape=jax.ShapeDtypeStruct((N_NODES, D), jnp.float32),
    )(x, nb0, nb1, cnt, wst, wnt0, wnt1, bias, gamma, beta)


@jax.jit
def kernel(x, edge_index, deg, W_self, W_neigh, bias, ln_gamma, ln_beta):
    del deg
    x0 = x[:, :DH].astype(jnp.bfloat16)
    x1 = x[:, DH:].astype(jnp.bfloat16)
    src2 = edge_index[1].reshape(ROWS, K)
    dst2 = edge_index[0].reshape(ROWS, K)
    dst1 = edge_index[0]
    nb0, nb1, cnt_tab = _sc_segment_sum(x0, x1, src2, dst2, dst1)
    cnt = cnt_tab.reshape(NPAD)[:N_NODES, None]
    wnt = W_neigh.T
    return _tc_combine(x, nb0, nb1, cnt, W_self.T, wnt[:DH], wnt[DH:],
                       bias[None, :], ln_gamma[None, :], ln_beta[None, :])


# speedup vs baseline: 1.0656x; 1.0656x over previous
import functools

import jax
import jax.numpy as jnp
from jax import lax
from jax.experimental import pallas as pl
from jax.experimental.pallas import tpu as pltpu
from jax.experimental.pallas import tpu_sc as plsc

N_NODES = 10000
NPAD = 10240
D = 256
DH = 128
E = 160000
K = 80
NS = 16
TROWS = E // K // NS
EPT = E // NS
NPT = N_NODES // NS
NBUF = 5
CROWS = NPAD // DH
OFFS = ((0, 80), (80, 80), (160, 80), (240, 80), (320, 80),
        (400, 80), (480, 80), (560, 65))


def _sc_segment_sum(x0, x1, src1, dst1):
    mesh = plsc.VectorSubcoreMesh(core_axis_name="c", subcore_axis_name="s")

    @functools.partial(
        pl.kernel,
        mesh=mesh,
        compiler_params=pltpu.CompilerParams(use_tc_tiling_on_sc=False,
                                             needs_layout_passes=False),
        out_type=(
            jax.ShapeDtypeStruct((N_NODES, DH), jnp.bfloat16),
            jax.ShapeDtypeStruct((N_NODES, DH), jnp.bfloat16),
            jax.ShapeDtypeStruct((CROWS, DH), jnp.float32),
        ),
        scratch_types=[
            pltpu.VMEM((EPT,), jnp.int32),
            pltpu.VMEM((EPT,), jnp.int32),
            [pltpu.VMEM((K, DH), jnp.bfloat16) for _ in range(NBUF)],
            pltpu.VMEM((CROWS, DH), jnp.float32),
            pltpu.VMEM((CROWS,), jnp.int32),
            pltpu.VMEM_SHARED((N_NODES, DH), jnp.bfloat16),
            pltpu.VMEM_SHARED((CROWS, DH), jnp.float32),
            [pltpu.SemaphoreType.DMA for _ in range(NBUF)],
            [pltpu.SemaphoreType.DMA for _ in range(NBUF)],
            pltpu.SemaphoreType.DMA,
            pltpu.SemaphoreType.DMA,
        ],
    )
    def k(x0_hbm, x1_hbm, src_hbm, dst_hbm, out0, out1, cnt_out,
          sidx, didx, bufs, blk, riota, acc, cnt_sp, semg, sems,
          semi, semw):
        c = lax.axis_index("c")
        s = lax.axis_index("s")

        zero16 = jnp.zeros((16,), jnp.float32)
        one16 = jnp.ones((16,), jnp.float32)
        zero32 = jnp.zeros((32,), jnp.bfloat16)

        pltpu.async_copy(src_hbm.at[pl.ds(s * EPT, EPT)], sidx, semi)
        pltpu.async_copy(dst_hbm.at[pl.ds(s * EPT, EPT)], didx, semi)

        rows0 = bufs[0]

        def zrow(i, carry):
            for j in range(DH // 32):
                rows0[i, pl.ds(j * 32, 32)] = zero32
            return carry

        lax.fori_loop(0, K, zrow, 0)

        def brow(i, carry):
            for j in range(DH // 16):
                blk[i, pl.ds(j * 16, 16)] = zero16
            return carry

        lax.fori_loop(0, CROWS, brow, 0)

        for j in range(CROWS // 16):
            riota[pl.ds(j * 16, 16)] = lax.iota(jnp.int32, 16) + j * 16

        for o, l in OFFS:
            pltpu.async_copy(rows0.at[pl.ds(0, l)],
                             acc.at[pl.ds(s * NPT + o, l)], semw)

        @pl.when(s == 0)
        def _():
            pltpu.async_copy(blk, cnt_sp, semw)

        for o, l in OFFS:
            pltpu.make_async_copy(rows0.at[pl.ds(0, l)],
                                  acc.at[pl.ds(s * NPT + o, l)], semw).wait()

        @pl.when(s == 0)
        def _():
            pltpu.make_async_copy(blk, cnt_sp, semw).wait()

        pltpu.make_async_copy(src_hbm.at[pl.ds(s * EPT, EPT)], sidx,
                              semi).wait()
        pltpu.make_async_copy(dst_hbm.at[pl.ds(s * EPT, EPT)], didx,
                              semi).wait()

        plsc.subcore_barrier()

        def edge_loop(x_ref):
            for i in range(NBUF):
                pltpu.async_copy(x_ref.at[sidx.at[pl.ds(i * K, K)]], bufs[i],
                                 semg[i])

            def cbody(j, carry):
                idx = didx[pl.ds(j * 16, 16)]
                plsc.addupdate_scatter(
                    blk,
                    [lax.shift_right_logical(idx, 7),
                     lax.bitwise_and(idx, 127)],
                    one16,
                )
                return carry

            lax.fori_loop(0, EPT // 16, cbody, 0)

            def step(q, carry):
                for i in range(NBUF):
                    a = NBUF * q + i
                    j = (i - 1) % NBUF
                    b = a - 1
                    pltpu.make_async_copy(x_ref.at[sidx.at[pl.ds(a * K, K)]],
                                          bufs[i], semg[i]).wait()

                    @pl.when(b >= 0)
                    def _():
                        pltpu.make_async_copy(
                            bufs[j], acc.at[didx.at[pl.ds(b * K, K)]],
                            sems[j]).wait()

                    pltpu.async_copy(bufs[i], acc.at[didx.at[pl.ds(a * K, K)]],
                                     sems[i], add=True)

                    @pl.when(jnp.logical_and(b >= 0, b + NBUF < TROWS))
                    def _():
                        pltpu.async_copy(
                            x_ref.at[sidx.at[pl.ds((b + NBUF) * K, K)]],
                            bufs[j], semg[j])
                return carry

            lax.fori_loop(0, TROWS // NBUF, step, 0)

            i = (TROWS - 1) % NBUF
            pltpu.make_async_copy(
                bufs[i], acc.at[didx.at[pl.ds((TROWS - 1) * K, K)]],
                sems[i]).wait()

        @pl.when(c == 0)
        def _():
            edge_loop(x0_hbm)

        @pl.when(c == 1)
        def _():
            edge_loop(x1_hbm)

        plsc.subcore_barrier()

        pltpu.sync_copy(blk, cnt_sp.at[riota], add=True)
        plsc.subcore_barrier()

        def copy_out(dref):
            for n, (o, l) in enumerate(OFFS):
                buf = bufs[n % 2]
                if n >= 2:
                    po, pll = OFFS[n - 2]
                    pltpu.make_async_copy(
                        buf.at[pl.ds(0, pll)],
                        dref.at[pl.ds(s * NPT + po, pll)], semw).wait()
                pltpu.sync_copy(acc.at[pl.ds(s * NPT + o, l)],
                                buf.at[pl.ds(0, l)])
                pltpu.async_copy(buf.at[pl.ds(0, l)],
                                 dref.at[pl.ds(s * NPT + o, l)], semw)
            for n in (len(OFFS) - 2, len(OFFS) - 1):
                o, l = OFFS[n]
                buf = bufs[n % 2]
                pltpu.make_async_copy(buf.at[pl.ds(0, l)],
                                      dref.at[pl.ds(s * NPT + o, l)],
                                      semw).wait()

        @pl.when(c == 0)
        def _():
            copy_out(out0)

        @pl.when(c == 1)
        def _():
            copy_out(out1)

        @pl.when(jnp.logical_and(c == 0, s == 0))
        def _():
            pltpu.sync_copy(cnt_sp, blk)
            pltpu.sync_copy(blk, cnt_out)

    return k(x0, x1, src1, dst1)


BM = 2000


def _tc_body(x_ref, nb0_ref, nb1_ref, cnt_ref, wst_ref, wnt0_ref, wnt1_ref,
             b_ref, g_ref, be_ref, o_ref):
    hs = jnp.dot(x_ref[...], wst_ref[...], preferred_element_type=jnp.float32)
    hn = (jnp.dot(nb0_ref[...].astype(jnp.float32), wnt0_ref[...],
                  preferred_element_type=jnp.float32)
          + jnp.dot(nb1_ref[...].astype(jnp.float32), wnt1_ref[...],
                    preferred_element_type=jnp.float32))
    inv = 1.0 / jnp.maximum(cnt_ref[...], 1.0)
    h = hs + hn * inv + b_ref[...]
    mu = jnp.mean(h, axis=-1, keepdims=True)
    d = h - mu
    var = jnp.mean(d * d, axis=-1, keepdims=True)
    o_ref[...] = d * lax.rsqrt(var + 1e-5) * g_ref[...] + be_ref[...]


def _tc_combine(x, nb0, nb1, cnt, wst, wnt0, wnt1, bias, gamma, beta):
    grid = (N_NODES // BM,)
    return pl.pallas_call(
        _tc_body,
        grid=grid,
        in_specs=[
            pl.BlockSpec((BM, D), lambda i: (i, 0)),
            pl.BlockSpec((BM, DH), lambda i: (i, 0)),
            pl.BlockSpec((BM, DH), lambda i: (i, 0)),
            pl.BlockSpec((BM, 1), lambda i: (i, 0)),
            pl.BlockSpec((D, D), lambda i: (0, 0)),
            pl.BlockSpec((DH, D), lambda i: (0, 0)),
            pl.BlockSpec((DH, D), lambda i: (0, 0)),
            pl.BlockSpec((1, D), lambda i: (0, 0)),
            pl.BlockSpec((1, D), lambda i: (0, 0)),
            pl.BlockSpec((1, D), lambda i: (0, 0)),
        ],
        out_specs=pl.BlockSpec((BM, D), lambda i: (i, 0)),
        out_shape=jax.ShapeDtypeStruct((N_NODES, D), jnp.float32),
    )(x, nb0, nb1, cnt, wst, wnt0, wnt1, bias, gamma, beta)


@jax.jit
def kernel(x, edge_index, deg, W_self, W_neigh, bias, ln_gamma, ln_beta):
    del deg
    x0 = x[:, :DH].astype(jnp.bfloat16)
    x1 = x[:, DH:].astype(jnp.bfloat16)
    src1 = edge_index[1]
    dst1 = edge_index[0]
    nb0, nb1, cnt_tab = _sc_segment_sum(x0, x1, src1, dst1)
    cnt = cnt_tab.reshape(NPAD)[:N_NODES, None]
    wnt = W_neigh.T
    return _tc_combine(x, nb0, nb1, cnt, W_self.T, wnt[:DH], wnt[DH:],
                       bias[None, :], ln_gamma[None, :], ln_beta[None, :])


# speedup vs baseline: 1.0695x; 1.0037x over previous
import functools

import jax
import jax.numpy as jnp
from jax import lax
from jax.experimental import pallas as pl
from jax.experimental.pallas import tpu as pltpu
from jax.experimental.pallas import tpu_sc as plsc

N_NODES = 10000
NPAD = 10240
D = 256
DH = 128
E = 160000
K = 80
NS = 16
TROWS = E // K // NS
EPT = E // NS
NPT = N_NODES // NS
NBUF = 5
CROWS = NPAD // DH
OFFS = ((0, 80), (80, 80), (160, 80), (240, 80), (320, 80),
        (400, 80), (480, 80), (560, 65))


def _sc_segment_sum(x0, x1, src1, dst1):
    mesh = plsc.VectorSubcoreMesh(core_axis_name="c", subcore_axis_name="s")

    @functools.partial(
        pl.kernel,
        mesh=mesh,
        compiler_params=pltpu.CompilerParams(use_tc_tiling_on_sc=False,
                                             needs_layout_passes=False),
        out_type=(
            jax.ShapeDtypeStruct((N_NODES, DH), jnp.bfloat16),
            jax.ShapeDtypeStruct((N_NODES, DH), jnp.bfloat16),
            jax.ShapeDtypeStruct((NPAD,), jnp.float32),
        ),
        scratch_types=[
            pltpu.VMEM((EPT,), jnp.int32),
            pltpu.VMEM((EPT,), jnp.int32),
            [pltpu.VMEM((K, DH), jnp.bfloat16) for _ in range(NBUF)],
            pltpu.VMEM((CROWS, DH), jnp.float32),
            pltpu.VMEM((CROWS,), jnp.int32),
            pltpu.VMEM_SHARED((N_NODES, DH), jnp.bfloat16),
            pltpu.VMEM_SHARED((CROWS, DH), jnp.float32),
            [pltpu.SemaphoreType.DMA for _ in range(NBUF)],
            [pltpu.SemaphoreType.DMA for _ in range(NBUF)],
            pltpu.SemaphoreType.DMA,
            pltpu.SemaphoreType.DMA,
        ],
    )
    def k(x0_hbm, x1_hbm, src_hbm, dst_hbm, out0, out1, cnt_out,
          sidx, didx, bufs, blk, riota, acc, cnt_sp, semg, sems,
          semi, semw):
        c = lax.axis_index("c")
        s = lax.axis_index("s")

        zero16 = jnp.zeros((16,), jnp.float32)
        one16 = jnp.ones((16,), jnp.float32)
        zero32 = jnp.zeros((32,), jnp.bfloat16)

        pltpu.async_copy(src_hbm.at[pl.ds(s * EPT, EPT)], sidx, semi)
        pltpu.async_copy(dst_hbm.at[pl.ds(s * EPT, EPT)], didx, semi)

        rows0 = bufs[0]

        def zrow(i, carry):
            for j in range(DH // 32):
                rows0[i, pl.ds(j * 32, 32)] = zero32
            return carry

        lax.fori_loop(0, K, zrow, 0)

        def brow(i, carry):
            for j in range(DH // 16):
                blk[i, pl.ds(j * 16, 16)] = zero16
            return carry

        lax.fori_loop(0, CROWS, brow, 0)

        for j in range(CROWS // 16):
            riota[pl.ds(j * 16, 16)] = lax.iota(jnp.int32, 16) + j * 16

        for o, l in OFFS:
            pltpu.async_copy(rows0.at[pl.ds(0, l)],
                             acc.at[pl.ds(s * NPT + o, l)], semw)

        @pl.when(s == 0)
        def _():
            pltpu.async_copy(blk, cnt_sp, semw)

        for o, l in OFFS:
            pltpu.make_async_copy(rows0.at[pl.ds(0, l)],
                                  acc.at[pl.ds(s * NPT + o, l)], semw).wait()

        @pl.when(s == 0)
        def _():
            pltpu.make_async_copy(blk, cnt_sp, semw).wait()

        pltpu.make_async_copy(src_hbm.at[pl.ds(s * EPT, EPT)], sidx,
                              semi).wait()
        pltpu.make_async_copy(dst_hbm.at[pl.ds(s * EPT, EPT)], didx,
                              semi).wait()

        plsc.subcore_barrier()

        def edge_loop(x_ref):
            for i in range(NBUF):
                pltpu.async_copy(x_ref.at[sidx.at[pl.ds(i * K, K)]], bufs[i],
                                 semg[i])

            def cbody(j, carry):
                idx = didx[pl.ds(j * 16, 16)]
                plsc.addupdate_scatter(
                    blk,
                    [lax.shift_right_logical(idx, 7),
                     lax.bitwise_and(idx, 127)],
                    one16,
                )
                return carry

            lax.fori_loop(0, EPT // 16, cbody, 0)

            def step(q, carry):
                for i in range(NBUF):
                    a = NBUF * q + i
                    j = (i - 1) % NBUF
                    b = a - 1
                    pltpu.make_async_copy(x_ref.at[sidx.at[pl.ds(a * K, K)]],
                                          bufs[i], semg[i]).wait()

                    @pl.when(b >= 0)
                    def _():
                        pltpu.make_async_copy(
                            bufs[j], acc.at[didx.at[pl.ds(b * K, K)]],
                            sems[j]).wait()

                    pltpu.async_copy(bufs[i], acc.at[didx.at[pl.ds(a * K, K)]],
                                     sems[i], add=True)

                    @pl.when(jnp.logical_and(b >= 0, b + NBUF < TROWS))
                    def _():
                        pltpu.async_copy(
                            x_ref.at[sidx.at[pl.ds((b + NBUF) * K, K)]],
                            bufs[j], semg[j])
                return carry

            lax.fori_loop(0, TROWS // NBUF, step, 0)

            i = (TROWS - 1) % NBUF
            pltpu.make_async_copy(
                bufs[i], acc.at[didx.at[pl.ds((TROWS - 1) * K, K)]],
                sems[i]).wait()

        @pl.when(c == 0)
        def _():
            edge_loop(x0_hbm)

        @pl.when(c == 1)
        def _():
            edge_loop(x1_hbm)

        plsc.subcore_barrier()

        pltpu.sync_copy(blk, cnt_sp.at[riota], add=True)
        plsc.subcore_barrier()

        def copy_out(dref):
            for n, (o, l) in enumerate(OFFS):
                buf = bufs[n % 2]
                if n >= 2:
                    po, pll = OFFS[n - 2]
                    pltpu.make_async_copy(
                        buf.at[pl.ds(0, pll)],
                        dref.at[pl.ds(s * NPT + po, pll)], semw).wait()
                pltpu.sync_copy(acc.at[pl.ds(s * NPT + o, l)],
                                buf.at[pl.ds(0, l)])
                pltpu.async_copy(buf.at[pl.ds(0, l)],
                                 dref.at[pl.ds(s * NPT + o, l)], semw)
            for n in (len(OFFS) - 2, len(OFFS) - 1):
                o, l = OFFS[n]
                buf = bufs[n % 2]
                pltpu.make_async_copy(buf.at[pl.ds(0, l)],
                                      dref.at[pl.ds(s * NPT + o, l)],
                                      semw).wait()

        @pl.when(c == 0)
        def _():
            copy_out(out0)

        @pl.when(c == 1)
        def _():
            copy_out(out1)

        @pl.when(c == 0)
        def _():
            pltpu.sync_copy(cnt_sp.at[pl.ds(5 * s, 5)], blk.at[pl.ds(0, 5)])
            for r in range(5):
                pltpu.async_copy(blk.at[r],
                                 cnt_out.at[pl.ds((5 * s + r) * DH, DH)],
                                 semw)
            for r in range(5):
                pltpu.make_async_copy(blk.at[r],
                                      cnt_out.at[pl.ds((5 * s + r) * DH, DH)],
                                      semw).wait()

    return k(x0, x1, src1, dst1)


BM = 2000


def _tc_body(x_ref, nb0_ref, nb1_ref, cnt_ref, wst_ref, wnt0_ref, wnt1_ref,
             b_ref, g_ref, be_ref, o_ref):
    hs = jnp.dot(x_ref[...], wst_ref[...], preferred_element_type=jnp.float32)
    hn = (jnp.dot(nb0_ref[...].astype(jnp.float32), wnt0_ref[...],
                  preferred_element_type=jnp.float32)
          + jnp.dot(nb1_ref[...].astype(jnp.float32), wnt1_ref[...],
                    preferred_element_type=jnp.float32))
    inv = 1.0 / jnp.maximum(cnt_ref[...], 1.0)
    h = hs + hn * inv + b_ref[...]
    mu = jnp.mean(h, axis=-1, keepdims=True)
    d = h - mu
    var = jnp.mean(d * d, axis=-1, keepdims=True)
    o_ref[...] = d * lax.rsqrt(var + 1e-5) * g_ref[...] + be_ref[...]


def _tc_combine(x, nb0, nb1, cnt, wst, wnt0, wnt1, bias, gamma, beta):
    grid = (N_NODES // BM,)
    return pl.pallas_call(
        _tc_body,
        grid=grid,
        in_specs=[
            pl.BlockSpec((BM, D), lambda i: (i, 0)),
            pl.BlockSpec((BM, DH), lambda i: (i, 0)),
            pl.BlockSpec((BM, DH), lambda i: (i, 0)),
            pl.BlockSpec((BM, 1), lambda i: (i, 0)),
            pl.BlockSpec((D, D), lambda i: (0, 0)),
            pl.BlockSpec((DH, D), lambda i: (0, 0)),
            pl.BlockSpec((DH, D), lambda i: (0, 0)),
            pl.BlockSpec((1, D), lambda i: (0, 0)),
            pl.BlockSpec((1, D), lambda i: (0, 0)),
            pl.BlockSpec((1, D), lambda i: (0, 0)),
        ],
        out_specs=pl.BlockSpec((BM, D), lambda i: (i, 0)),
        out_shape=jax.ShapeDtypeStruct((N_NODES, D), jnp.float32),
    )(x, nb0, nb1, cnt, wst, wnt0, wnt1, bias, gamma, beta)


@jax.jit
def kernel(x, edge_index, deg, W_self, W_neigh, bias, ln_gamma, ln_beta):
    del deg
    x0 = x[:, :DH].astype(jnp.bfloat16)
    x1 = x[:, DH:].astype(jnp.bfloat16)
    src1 = edge_index[1]
    dst1 = edge_index[0]
    nb0, nb1, cnt1d = _sc_segment_sum(x0, x1, src1, dst1)
    cnt = cnt1d[:N_NODES, None]
    wnt = W_neigh.T
    return _tc_combine(x, nb0, nb1, cnt, W_self.T, wnt[:DH], wnt[DH:],
                       bias[None, :], ln_gamma[None, :], ln_beta[None, :])
